# baseline (speedup 1.0000x reference)
import functools

import jax
import jax.numpy as jnp
from jax.experimental import pallas as pl
from jax.experimental.pallas import tpu as pltpu

B = 64
T = 200
AD = 128
ED = 128
OD = 2048


def _fused_kernel(a_feat_ref, v_feat_ref, mask_a_ref, mask_v_ref,
                  modal_ref, time_ref, pad_ref,
                  Wa_ref, ba_ref, Wv_ref, bv_ref,
                  out_ref,
                  base_a_ref, base_v_ref, wpad_a_ref, wpad_v_ref):
    b = pl.program_id(0)

    @pl.when(b == 0)
    def _():
        te = time_ref[...]
        ea = te + modal_ref[0:1, :]
        ev = te + modal_ref[1:2, :]
        Wae = Wa_ref[:, AD:]
        Wve = Wv_ref[:, AD:]
        dn = (((1,), (1,)), ((), ()))
        base_a_ref[...] = (
            jax.lax.dot_general(Wae, ea, dn, preferred_element_type=jnp.float32)
            + ba_ref[...])
        base_v_ref[...] = (
            jax.lax.dot_general(Wve, ev, dn, preferred_element_type=jnp.float32)
            + bv_ref[...])
        wpad_a_ref[...] = jax.lax.dot_general(
            Wae, pad_ref[...], dn, preferred_element_type=jnp.float32)
        wpad_v_ref[...] = jax.lax.dot_general(
            Wve, pad_ref[...], dn, preferred_element_type=jnp.float32)

    dn = (((1,), (1,)), ((), ()))
    a_res = (
        jax.lax.dot_general(Wa_ref[:, :AD], a_feat_ref[0], dn,
                            preferred_element_type=jnp.float32)
        + base_a_ref[...]
        + wpad_a_ref[...] * mask_a_ref[0])
    v_res = (
        jax.lax.dot_general(Wv_ref[:, :AD], v_feat_ref[0], dn,
                            preferred_element_type=jnp.float32)
        + base_v_ref[...]
        + wpad_v_ref[...] * mask_v_ref[0])
    out_ref[0, :, :T] = a_res
    out_ref[0, :, T:] = v_res


@jax.jit
def kernel(audio_feat, video_feat, mask_audio, mask_video, modal_emb,
           time_emb, pad_emb, W_audio, b_audio, W_video, b_video):
    mask_a = mask_audio.astype(jnp.float32).reshape(B, 1, T)
    mask_v = mask_video.astype(jnp.float32).reshape(B, 1, T)
    ba = b_audio.reshape(OD, 1)
    bv = b_video.reshape(OD, 1)

    grid = (B,)
    out = pl.pallas_call(
        _fused_kernel,
        grid=grid,
        in_specs=[
            pl.BlockSpec((1, T, AD), lambda b: (b, 0, 0)),
            pl.BlockSpec((1, T, AD), lambda b: (b, 0, 0)),
            pl.BlockSpec((1, 1, T), lambda b: (b, 0, 0)),
            pl.BlockSpec((1, 1, T), lambda b: (b, 0, 0)),
            pl.BlockSpec((2, ED), lambda b: (0, 0)),
            pl.BlockSpec((T, ED), lambda b: (0, 0)),
            pl.BlockSpec((1, ED), lambda b: (0, 0)),
            pl.BlockSpec((OD, AD + ED), lambda b: (0, 0)),
            pl.BlockSpec((OD, 1), lambda b: (0, 0)),
            pl.BlockSpec((OD, AD + ED), lambda b: (0, 0)),
            pl.BlockSpec((OD, 1), lambda b: (0, 0)),
        ],
        out_specs=pl.BlockSpec((1, OD, 2 * T), lambda b: (b, 0, 0)),
        out_shape=jax.ShapeDtypeStruct((B, OD, 2 * T), jnp.float32),
        scratch_shapes=[
            pltpu.VMEM((OD, T), jnp.float32),
            pltpu.VMEM((OD, T), jnp.float32),
            pltpu.VMEM((OD, 1), jnp.float32),
            pltpu.VMEM((OD, 1), jnp.float32),
        ],
    )(audio_feat, video_feat, mask_a, mask_v, modal_emb, time_emb, pad_emb,
      W_audio, ba, W_video, bv)
    return out
